# Initial kernel scaffold; baseline (speedup 1.0000x reference)
import jax
import jax.numpy as jnp
from jax.experimental import pallas as pl


def kernel(x, edge_index, edge_weight, tau, W_self, W_neigh, b_conv, W_mlp1, b_mlp1, W_mlp2, b_mlp2):
    raise NotImplementedError("write your pallas kernel here")


# speedup vs baseline: 4.5248x; 4.5248x over previous
import functools

import jax
import jax.numpy as jnp
from jax import lax
from jax.experimental import pallas as pl
from jax.experimental.pallas import tpu as pltpu
from jax.experimental.pallas import tpu_sc as plsc

NC = 2
NS = 16
L = 16
NW = NC * NS
CHUNK = 128


def _zero_fill(ref, nrows, ncols):
    def body(i, c):
        for j in range(ncols // L):
            ref[i, pl.ds(j * L, L)] = jnp.zeros((L,), jnp.float32)
        return c
    lax.fori_loop(0, nrows, body, 0)


def _spmem_init(zbuf, acc, s, rpt, sem):
    for t in range(rpt // CHUNK):
        pltpu.async_copy(
            zbuf, acc.at[pl.ds(s * rpt + t * CHUNK, CHUNK)], sem).wait()


def _staged_writeout(acc, stage, out_slice, s, rpt, sem):
    for t in range(rpt // CHUNK):
        base = s * rpt + t * CHUNK
        pltpu.async_copy(acc.at[pl.ds(base, CHUNK)], stage, sem).wait()
        pltpu.sync_copy(stage, out_slice.at[pl.ds(base, CHUNK)])


def _partition(N, E):
    blk = NS * CHUNK
    n_pad = ((N + blk - 1) // blk) * blk
    n_chunks = E // CHUNK
    return dict(
        base_chunks=n_chunks // NW,
        extra=n_chunks % NW,
        n_pad=n_pad,
        rpt=n_pad // NS,
    )


def _sc_round1(x, src, dst, wrep):
    N, D = x.shape
    E = src.shape[0]
    p = _partition(N, E)
    base_chunks, extra = p["base_chunks"], p["extra"]
    n_pad, rpt = p["n_pad"], p["rpt"]

    @functools.partial(
        pl.kernel,
        out_type=jax.ShapeDtypeStruct((NC, n_pad, D), jnp.float32),
        mesh=plsc.VectorSubcoreMesh(core_axis_name="c", subcore_axis_name="s",
                                    num_cores=NC),
        scratch_types=[
            pltpu.VMEM((CHUNK,), jnp.int32),
            pltpu.VMEM((CHUNK,), jnp.int32),
            pltpu.VMEM((CHUNK, L), jnp.float32),
            pltpu.VMEM((CHUNK, D), jnp.float32),
            pltpu.VMEM_SHARED((n_pad, D), jnp.float32),
            pltpu.SemaphoreType.DMA,
        ],
    )
    def k(x_hbm, src_hbm, dst_hbm, w_hbm, out_hbm,
          srcv, dstv, wvm, rowsv, acc, sem):
        c = lax.axis_index("c")
        s = lax.axis_index("s")
        wid = s * NC + c

        _zero_fill(rowsv, CHUNK, D)
        _spmem_init(rowsv, acc, s, rpt, sem)
        plsc.subcore_barrier()

        nk = base_chunks + jnp.where(wid < extra, 1, 0)

        def chunk_body(kk, carry):
            base = (wid + NW * kk) * CHUNK
            pltpu.sync_copy(src_hbm.at[pl.ds(base, CHUNK)], srcv)
            pltpu.sync_copy(dst_hbm.at[pl.ds(base, CHUNK)], dstv)
            pltpu.sync_copy(w_hbm.at[pl.ds(base, CHUNK)], wvm)
            pltpu.async_copy(x_hbm.at[srcv], rowsv, sem).wait()

            def edge_body(e, c2):
                wb = wvm[e, :]
                for j in range(D // L):
                    rowsv[e, pl.ds(j * L, L)] = rowsv[e, pl.ds(j * L, L)] * wb
                return c2

            lax.fori_loop(0, CHUNK, edge_body, 0)
            pltpu.async_copy(rowsv, acc.at[dstv], sem, add=True).wait()
            return carry

        lax.fori_loop(0, nk, chunk_body, 0)
        plsc.subcore_barrier()
        _staged_writeout(acc, rowsv, out_hbm.at[c], s, rpt, sem)

    return k(x, src, dst, wrep)


def _sc_round2(q, src, dst):
    N, D = q.shape
    E = src.shape[0]
    p = _partition(N, E)
    base_chunks, extra = p["base_chunks"], p["extra"]
    n_pad, rpt = p["n_pad"], p["rpt"]

    @functools.partial(
        pl.kernel,
        out_type=jax.ShapeDtypeStruct((NC, n_pad, D), jnp.float32),
        mesh=plsc.VectorSubcoreMesh(core_axis_name="c", subcore_axis_name="s",
                                    num_cores=NC),
        scratch_types=[
            pltpu.VMEM((CHUNK,), jnp.int32),
            pltpu.VMEM((CHUNK,), jnp.int32),
            pltpu.VMEM((CHUNK, D), jnp.float32),
            pltpu.VMEM((CHUNK, L), jnp.float32),
            pltpu.VMEM_SHARED((n_pad, D), jnp.float32),
            pltpu.SemaphoreType.DMA,
        ],
    )
    def k(q_hbm, src_hbm, dst_hbm, out_hbm,
          srcv, dstv, rowsv, onesv, acc, sem):
        c = lax.axis_index("c")
        s = lax.axis_index("s")
        wid = s * NC + c

        _zero_fill(rowsv, CHUNK, D)
        _zero_fill(onesv, CHUNK, L)
        _spmem_init(rowsv, acc, s, rpt, sem)
        plsc.subcore_barrier()

        nk = base_chunks + jnp.where(wid < extra, 1, 0)

        def chunk_body(kk, carry):
            base = (wid + NW * kk) * CHUNK
            pltpu.sync_copy(src_hbm.at[pl.ds(base, CHUNK)], srcv)
            pltpu.sync_copy(dst_hbm.at[pl.ds(base, CHUNK)], dstv)
            pltpu.async_copy(q_hbm.at[srcv], rowsv, sem).wait()
            pltpu.async_copy(rowsv, acc.at[dstv], sem, add=True).wait()
            return carry

        lax.fori_loop(0, nk, chunk_body, 0)
        plsc.subcore_barrier()
        _staged_writeout(acc, rowsv, out_hbm.at[c], s, rpt, sem)

    return k(q, src, dst)


def _sc_deg(dst, n_nodes, width):
    E = dst.shape[0]
    D = width
    p = _partition(n_nodes, E)
    base_chunks, extra = p["base_chunks"], p["extra"]
    n_pad, rpt = p["n_pad"], p["rpt"]

    @functools.partial(
        pl.kernel,
        out_type=jax.ShapeDtypeStruct((NC, n_pad, D), jnp.float32),
        mesh=plsc.VectorSubcoreMesh(core_axis_name="c", subcore_axis_name="s",
                                    num_cores=NC),
        scratch_types=[
            pltpu.VMEM((CHUNK,), jnp.int32),
            pltpu.VMEM((CHUNK, D), jnp.float32),
            pltpu.VMEM_SHARED((n_pad, D), jnp.float32),
            pltpu.SemaphoreType.DMA,
        ],
    )
    def k(dst_hbm, deg_hbm, dstv, onesv, dacc, sem):
        c = lax.axis_index("c")
        s = lax.axis_index("s")
        wid = s * NC + c

        _zero_fill(onesv, CHUNK, D)
        _spmem_init(onesv, dacc, s, rpt, sem)

        def ones_body(i, c2):
            for j in range(D // L):
                onesv[i, pl.ds(j * L, L)] = jnp.full((L,), 1.0, jnp.float32)
            return c2
        lax.fori_loop(0, CHUNK, ones_body, 0)
        plsc.subcore_barrier()

        nk = base_chunks + jnp.where(wid < extra, 1, 0)

        def chunk_body(kk, carry):
            base = (wid + NW * kk) * CHUNK
            pltpu.sync_copy(dst_hbm.at[pl.ds(base, CHUNK)], dstv)
            pltpu.async_copy(onesv, dacc.at[dstv], sem, add=True).wait()
            return carry

        lax.fori_loop(0, nk, chunk_body, 0)
        plsc.subcore_barrier()
        _staged_writeout(dacc, onesv, deg_hbm.at[c], s, rpt, sem)

    return k(dst)


def _tc_stage_b(tau1, x, k0, k1, wn_top, wn_bot):
    N, D = x.shape
    H = wn_top.shape[1]
    BLK = 1000

    def body(tau_ref, x_ref, k0_ref, k1_ref, wt_ref, wb_ref, q_ref, kx_ref):
        sig = 1.0 / (1.0 + jnp.exp(-tau_ref[0]))
        kx = sig * (k0_ref[...] + k1_ref[...])
        kx_ref[...] = kx
        q_ref[...] = (
            sig * jnp.dot(x_ref[...], wt_ref[...], preferred_element_type=jnp.float32)
            + jnp.dot(kx, wb_ref[...], preferred_element_type=jnp.float32)
        )

    return pl.pallas_call(
        body,
        grid=(N // BLK,),
        in_specs=[
            pl.BlockSpec(memory_space=pltpu.SMEM),
            pl.BlockSpec((BLK, D), lambda i: (i, 0)),
            pl.BlockSpec((BLK, D), lambda i: (i, 0)),
            pl.BlockSpec((BLK, D), lambda i: (i, 0)),
            pl.BlockSpec((D, H), lambda i: (0, 0)),
            pl.BlockSpec((D, H), lambda i: (0, 0)),
        ],
        out_specs=[
            pl.BlockSpec((BLK, H), lambda i: (i, 0)),
            pl.BlockSpec((BLK, D), lambda i: (i, 0)),
        ],
        out_shape=[
            jax.ShapeDtypeStruct((N, H), jnp.float32),
            jax.ShapeDtypeStruct((N, D), jnp.float32),
        ],
    )(tau1, x, k0, k1, wn_top, wn_bot)


def _tc_stage_c(tau1, x, kx, s0, s1, d0, d1,
                ws_top, ws_bot, b_conv, w1, b1, w2, b2):
    N, D = x.shape
    H = ws_top.shape[1]
    OUT = w2.shape[1]
    BLK = 1000

    def body(tau_ref, x_ref, kx_ref, s0_ref, s1_ref, d0_ref, d1_ref,
             wst_ref, wsb_ref, bc_ref, w1_ref, b1_ref, w2_ref, b2_ref, z_ref):
        sig = 1.0 / (1.0 + jnp.exp(-tau_ref[0]))
        deg = d0_ref[...][:, :1] + d1_ref[...][:, :1]
        svals = (s0_ref[...] + s1_ref[...]) / jnp.maximum(deg, 1.0)
        pre = (
            sig * jnp.dot(x_ref[...], wst_ref[...], preferred_element_type=jnp.float32)
            + jnp.dot(kx_ref[...], wsb_ref[...], preferred_element_type=jnp.float32)
            + svals + bc_ref[...]
        )
        h = jnp.maximum(pre, 0.0)
        h2 = jnp.maximum(
            jnp.dot(h, w1_ref[...], preferred_element_type=jnp.float32) + b1_ref[...],
            0.0,
        )
        z_ref[...] = (
            jnp.dot(h2, w2_ref[...], preferred_element_type=jnp.float32) + b2_ref[...]
        )

    row = lambda i: (i, 0)
    fixed = lambda i: (0, 0)
    return pl.pallas_call(
        body,
        grid=(N // BLK,),
        in_specs=[
            pl.BlockSpec(memory_space=pltpu.SMEM),
            pl.BlockSpec((BLK, D), row),
            pl.BlockSpec((BLK, D), row),
            pl.BlockSpec((BLK, H), row),
            pl.BlockSpec((BLK, H), row),
            pl.BlockSpec((BLK, L), row),
            pl.BlockSpec((BLK, L), row),
            pl.BlockSpec((D, H), fixed),
            pl.BlockSpec((D, H), fixed),
            pl.BlockSpec((1, H), fixed),
            pl.BlockSpec((H, H), fixed),
            pl.BlockSpec((1, H), fixed),
            pl.BlockSpec((H, OUT), fixed),
            pl.BlockSpec((1, OUT), fixed),
        ],
        out_specs=pl.BlockSpec((BLK, OUT), row),
        out_shape=jax.ShapeDtypeStruct((N, OUT), jnp.float32),
    )(tau1, x, kx, s0, s1, d0, d1, ws_top, ws_bot, b_conv, w1, b1, w2, b2)


def kernel(x, edge_index, edge_weight, tau, W_self, W_neigh, b_conv,
           W_mlp1, b_mlp1, W_mlp2, b_mlp2):
    N, D = x.shape
    src = edge_index[0]
    dst = edge_index[1]

    tau1 = jnp.reshape(tau, (1,)).astype(jnp.float32)
    wrep = jnp.broadcast_to(edge_weight[:, None], (edge_weight.shape[0], L))

    kxp = _sc_round1(x, src, dst, wrep)[:, :N]

    ws_top, ws_bot = W_self[:D], W_self[D:]
    wn_top, wn_bot = W_neigh[:D], W_neigh[D:]

    q, kx = _tc_stage_b(tau1, x, kxp[0], kxp[1], wn_top, wn_bot)
    sp = _sc_round2(q, src, dst)[:, :N]
    degp = _sc_deg(dst, N, D)[:, :N, :L]
    z = _tc_stage_c(tau1, x, kx, sp[0], sp[1], degp[0], degp[1],
                    ws_top, ws_bot, b_conv.reshape(1, -1),
                    W_mlp1, b_mlp1.reshape(1, -1),
                    W_mlp2, b_mlp2.reshape(1, -1))
    return z
